# baseline (speedup 1.0000x reference)
import functools

import jax
import jax.numpy as jnp
from jax import lax
from jax.experimental import pallas as pl
from jax.experimental.pallas import tpu as pltpu
from jax.experimental.pallas import tpu_sc as plsc

E = 64
TR = 128
NP = 4096
NPT = NP // TR
NSMAX = 96
CH = 128
NC = 2
NS = 16
NW = NC * NS
L = 16


def _router_meta_kernel(x_ref, wr_ref, wb_ref, p0_ref, p1_ref, tile_ref,
                        ex_ref, lo_ref, hi_ref, nt_ref):
    S = x_ref.shape[0]
    xf = x_ref[...]
    logits = jnp.dot(xf, wr_ref[...], preferred_element_type=jnp.float32)
    probs = jax.nn.softmax(logits, axis=-1)
    eiota = lax.broadcasted_iota(jnp.int32, (S, E), 1)
    m1 = jnp.max(probs, axis=1, keepdims=True)
    i1 = jnp.min(jnp.where(probs == m1, eiota, E), axis=1, keepdims=True)
    mask1 = eiota == i1
    probs2 = jnp.where(mask1, -1.0, probs)
    m2 = jnp.max(probs2, axis=1, keepdims=True)
    i2 = jnp.min(jnp.where(probs2 == m2, eiota, E), axis=1, keepdims=True)
    denom = m1 + m2
    wb_ref[...] = jnp.concatenate(
        [jnp.broadcast_to(m1 / denom, (S, L)),
         jnp.broadcast_to(m2 / denom, (S, L))], axis=1)

    oh0 = mask1.astype(jnp.float32)
    oh1 = (eiota == i2).astype(jnp.float32)
    counts = jnp.sum(oh0 + oh1, axis=0, keepdims=True)
    ri = lax.broadcasted_iota(jnp.int32, (E, E), 0)
    ci = lax.broadcasted_iota(jnp.int32, (E, E), 1)
    upper = (ri < ci).astype(jnp.float32)
    offs = jnp.dot(counts, upper,
                   preferred_element_type=jnp.float32)
    erow = lax.broadcasted_iota(jnp.int32, (1, E), 1)
    laste = jnp.max(jnp.where(counts > 0.0, erow, 0))

    rl = lax.broadcasted_iota(jnp.int32, (CH, CH), 0)
    cl = lax.broadcasted_iota(jnp.int32, (CH, CH), 1)
    lstrict = (cl < rl).astype(jnp.float32)
    base = jnp.zeros((1, E), jnp.float32)
    for c in range(S // CH):
        sl = slice(c * CH, (c + 1) * CH)
        o0 = oh0[sl]
        o1 = oh1[sl]
        poff = jnp.dot(lstrict, o0 + o1,
                       preferred_element_type=jnp.float32) + base + offs
        pos0 = jnp.sum(o0 * poff, axis=1, keepdims=True)
        pos1 = jnp.sum(o1 * poff, axis=1, keepdims=True)
        p0_ref[sl, :] = jnp.broadcast_to(pos0, (CH, 8)).astype(jnp.int32)
        p1_ref[sl, :] = jnp.broadcast_to(pos1, (CH, 8)).astype(jnp.int32)
        base = base + jnp.sum(o0 + o1, axis=0, keepdims=True)

    ntl = jnp.where(counts > 0.0,
                    jnp.floor((offs + counts - 1.0) / TR)
                    - jnp.floor(offs / TR) + 1.0, 0.0)
    cumt = jnp.dot(ntl, upper, preferred_element_type=jnp.float32) + ntl
    cumt_ex = cumt - ntl
    nsteps = jnp.sum(ntl)
    kio = lax.broadcasted_iota(jnp.int32, (128, E), 0).astype(
        jnp.float32)
    se = jnp.sum((jnp.broadcast_to(cumt, (128, E)) <= kio).astype(jnp.int32),
                 axis=1, keepdims=True)
    se = jnp.minimum(se, laste)
    ohse = (lax.broadcasted_iota(jnp.int32, (128, E), 1) == se).astype(
        jnp.float32)
    off_k = jnp.sum(ohse * offs, axis=1, keepdims=True)
    c_k = jnp.sum(ohse * counts, axis=1, keepdims=True)
    cex_k = jnp.sum(ohse * cumt_ex, axis=1, keepdims=True)
    kcol = lax.broadcasted_iota(jnp.int32, (128, 1), 0).astype(jnp.float32)
    tile_k = jnp.clip(jnp.floor(off_k / TR) + (kcol - cex_k), 0.0,
                      float(NPT - 1))
    lo_k = jnp.maximum(off_k - tile_k * TR, 0.0)
    hi_k = jnp.minimum(off_k + c_k - tile_k * TR, float(TR))
    tile_ref[...] = jnp.broadcast_to(tile_k, (128, 8)).astype(jnp.int32)
    ex_ref[...] = jnp.broadcast_to(se, (128, 8)).astype(jnp.int32)
    lo_ref[...] = jnp.broadcast_to(lo_k, (128, 8)).astype(jnp.int32)
    hi_ref[...] = jnp.broadcast_to(hi_k, (128, 8)).astype(jnp.int32)
    nt_ref[...] = jnp.full((8, 128), nsteps.astype(jnp.int32), jnp.int32)


def _gmm_kernel(tile_ref, ex_ref, lo_ref, hi_ref, ns_ref, xs_ref, w1_ref,
                w2_ref, ys_ref):
    k = pl.program_id(0)

    @pl.when(k < ns_ref[0])
    def _():
        h = jnp.dot(xs_ref[...], w1_ref[0], preferred_element_type=jnp.float32)
        h = h * jax.nn.sigmoid(h)
        y = jnp.dot(h, w2_ref[0], preferred_element_type=jnp.float32)
        rowi = lax.broadcasted_iota(jnp.int32, (y.shape[0], 1), 0)
        mask = (rowi >= lo_ref[k]) & (rowi < hi_ref[k])
        ys_ref[...] = jnp.where(mask, y, ys_ref[...])


def _make_sc_kernels(S, D):
    TPW = S // NW
    HW = TPW // 2
    QW = TPW // 4
    mesh = plsc.VectorSubcoreMesh(core_axis_name="c", subcore_axis_name="s",
                                  num_cores=NC)

    @functools.partial(
        pl.kernel, mesh=mesh,
        out_type=jax.ShapeDtypeStruct((NP, D), jnp.float32),
        scratch_types=[
            pltpu.VMEM((HW,), jnp.int32),
            pltpu.VMEM((HW,), jnp.int32),
            pltpu.VMEM((HW,), jnp.int32),
            pltpu.VMEM((HW,), jnp.int32),
            pltpu.VMEM((HW, D), jnp.float32),
            pltpu.VMEM((HW, D), jnp.float32),
            pltpu.SemaphoreType.DMA,
            pltpu.SemaphoreType.DMA,
        ],
    )
    def dispatch(x_hbm, posr_hbm, xs_hbm, idx0a_v, idx1a_v, idx0b_v, idx1b_v,
                 rows_a, rows_b, sem_in, sem_out):
        wid = lax.axis_index("s") * NC + lax.axis_index("c")
        base = wid * TPW
        cpa = pltpu.async_copy(x_hbm.at[pl.ds(base, HW)], rows_a, sem_in)
        cpb = pltpu.async_copy(x_hbm.at[pl.ds(base + HW, HW)], rows_b, sem_in)
        pltpu.sync_copy(posr_hbm.at[wid, 0, pl.ds(0, HW)], idx0a_v)
        pltpu.sync_copy(posr_hbm.at[wid, 1, pl.ds(0, HW)], idx1a_v)
        pltpu.sync_copy(posr_hbm.at[wid, 0, pl.ds(HW, HW)], idx0b_v)
        pltpu.sync_copy(posr_hbm.at[wid, 1, pl.ds(HW, HW)], idx1b_v)
        cpa.wait()
        s0a = pltpu.async_copy(rows_a, xs_hbm.at[idx0a_v], sem_out)
        s1a = pltpu.async_copy(rows_a, xs_hbm.at[idx1a_v], sem_out)
        cpb.wait()
        s0b = pltpu.async_copy(rows_b, xs_hbm.at[idx0b_v], sem_out)
        s1b = pltpu.async_copy(rows_b, xs_hbm.at[idx1b_v], sem_out)
        s0a.wait()
        s1a.wait()
        s0b.wait()
        s1b.wait()

    @functools.partial(
        pl.kernel, mesh=mesh,
        out_type=jax.ShapeDtypeStruct((S, D), jnp.float32),
        scratch_types=[
            pltpu.VMEM((TPW,), jnp.int32),
            pltpu.VMEM((TPW,), jnp.int32),
            pltpu.VMEM((TPW, 2 * L), jnp.float32),
            pltpu.VMEM((QW, D), jnp.float32),
            pltpu.VMEM((QW, D), jnp.float32),
            pltpu.VMEM((QW, D), jnp.float32),
            pltpu.VMEM((QW, D), jnp.float32),
            pltpu.VMEM((QW, D), jnp.float32),
            pltpu.SemaphoreType.DMA,
        ],
    )
    def combine(ys_hbm, posr_hbm, wb_hbm, out_hbm,
                idx0_v, idx1_v, wb_v, yg0a, yg1a, yg0b, yg1b, out_v, sem):
        wid = lax.axis_index("s") * NC + lax.axis_index("c")
        base = wid * TPW
        pltpu.sync_copy(wb_hbm.at[pl.ds(base, TPW)], wb_v)
        pltpu.sync_copy(posr_hbm.at[wid, 0], idx0_v)
        pltpu.sync_copy(posr_hbm.at[wid, 1], idx1_v)
        bufs = [(yg0a, yg1a), (yg0b, yg1b)]

        def fire(sub):
            g0, g1 = bufs[sub % 2]
            s0 = sub * QW
            c0 = pltpu.async_copy(ys_hbm.at[idx0_v.at[pl.ds(s0, QW)]], g0, sem)
            c1 = pltpu.async_copy(ys_hbm.at[idx1_v.at[pl.ds(s0, QW)]], g1, sem)
            return c0, c1

        pend = fire(0)
        for sub in range(TPW // QW):
            s0 = sub * QW
            g0, g1 = bufs[sub % 2]
            pend[0].wait()
            pend[1].wait()
            if sub < TPW // QW - 1:
                pend = fire(sub + 1)

            def body(t, carry, s0=s0, g0=g0, g1=g1):
                w0 = wb_v[s0 + t, pl.ds(0, L)]
                w1 = wb_v[s0 + t, pl.ds(L, L)]
                for s2 in range(D // L):
                    slc = pl.ds(s2 * L, L)
                    out_v[t, slc] = w0 * g0[t, slc] + w1 * g1[t, slc]
                return carry

            lax.fori_loop(0, QW, body, 0)
            pltpu.sync_copy(out_v, out_hbm.at[pl.ds(base + s0, QW)])

    return dispatch, combine


def kernel(x, Wr, W1, W2):
    b, s, d = x.shape
    e, _, h = W1.shape
    xf = x.reshape(s, d)

    tpw = s // NW
    wb, p0o, p1o, tileo, exo, loo, hio, nto = pl.pallas_call(
        _router_meta_kernel,
        out_shape=[
            jax.ShapeDtypeStruct((s, 2 * L), jnp.float32),
            jax.ShapeDtypeStruct((s, 8), jnp.int32),
            jax.ShapeDtypeStruct((s, 8), jnp.int32),
            jax.ShapeDtypeStruct((128, 8), jnp.int32),
            jax.ShapeDtypeStruct((128, 8), jnp.int32),
            jax.ShapeDtypeStruct((128, 8), jnp.int32),
            jax.ShapeDtypeStruct((128, 8), jnp.int32),
            jax.ShapeDtypeStruct((8, 128), jnp.int32),
        ],
    )(xf, Wr)

    posr = jnp.stack(
        [p0o[:, 0].reshape(NW, tpw), p1o[:, 0].reshape(NW, tpw)], axis=1)
    tile_a = tileo[:NSMAX, 0]
    ex_a = exo[:NSMAX, 0]
    lo_a = loo[:NSMAX, 0]
    hi_a = hio[:NSMAX, 0]
    ns_a = nto[0, :1]

    dispatch, combine = _make_sc_kernels(s, d)
    xs = dispatch(xf, posr)

    grid_spec = pltpu.PrefetchScalarGridSpec(
        num_scalar_prefetch=5,
        grid=(NSMAX,),
        in_specs=[
            pl.BlockSpec((TR, d), lambda k, t_r, e_r, l_r, h_r, n_r:
                         (t_r[k], 0)),
            pl.BlockSpec((1, d, h), lambda k, t_r, e_r, l_r, h_r, n_r:
                         (e_r[k], 0, 0)),
            pl.BlockSpec((1, h, d), lambda k, t_r, e_r, l_r, h_r, n_r:
                         (e_r[k], 0, 0)),
        ],
        out_specs=pl.BlockSpec((TR, d), lambda k, t_r, e_r, l_r, h_r, n_r:
                               (t_r[k], 0)),
    )
    ys = pl.pallas_call(
        _gmm_kernel,
        grid_spec=grid_spec,
        out_shape=jax.ShapeDtypeStruct((NP, d), jnp.float32),
    )(tile_a, ex_a, lo_a, hi_a, ns_a, xs, W1, W2)

    out = combine(ys, posr, wb)
    return out.reshape(b, s, d)


# speedup vs baseline: 1.1804x; 1.1804x over previous
import functools

import jax
import jax.numpy as jnp
from jax import lax
from jax.experimental import pallas as pl
from jax.experimental.pallas import tpu as pltpu
from jax.experimental.pallas import tpu_sc as plsc

E = 64
TR = 128
NT = 96
NP = NT * TR
CH = 128
NC = 2
NS = 16
NW = NC * NS
L = 16


def _router_meta_kernel(x_ref, wr_ref, wb_ref, p0_ref, p1_ref, et_ref, nt_ref):
    S = x_ref.shape[0]
    xf = x_ref[...]
    logits = jnp.dot(xf, wr_ref[...], preferred_element_type=jnp.float32)
    probs = jax.nn.softmax(logits, axis=-1)
    eiota = lax.broadcasted_iota(jnp.int32, (S, E), 1)
    m1 = jnp.max(probs, axis=1, keepdims=True)
    i1 = jnp.min(jnp.where(probs == m1, eiota, E), axis=1, keepdims=True)
    mask1 = eiota == i1
    probs2 = jnp.where(mask1, -1.0, probs)
    m2 = jnp.max(probs2, axis=1, keepdims=True)
    i2 = jnp.min(jnp.where(probs2 == m2, eiota, E), axis=1, keepdims=True)
    denom = m1 + m2
    wb_ref[...] = jnp.concatenate(
        [jnp.broadcast_to(m1 / denom, (S, L)),
         jnp.broadcast_to(m2 / denom, (S, L))], axis=1)

    oh0 = mask1.astype(jnp.float32)
    oh1 = (eiota == i2).astype(jnp.float32)
    counts = jnp.sum(oh0 + oh1, axis=0, keepdims=True)
    pc = jnp.ceil(counts / TR) * TR
    ri = lax.broadcasted_iota(jnp.int32, (E, E), 0)
    ci = lax.broadcasted_iota(jnp.int32, (E, E), 1)
    upper = (ri < ci).astype(jnp.float32)
    offs = jnp.dot(pc, upper, preferred_element_type=jnp.float32)
    ends = offs + pc
    total = jnp.sum(pc)
    erow = lax.broadcasted_iota(jnp.int32, (1, E), 1)
    laste = jnp.max(jnp.where(counts > 0.0, erow, 0))

    rl = lax.broadcasted_iota(jnp.int32, (CH, CH), 0)
    cl = lax.broadcasted_iota(jnp.int32, (CH, CH), 1)
    lstrict = (cl < rl).astype(jnp.float32)
    base = jnp.zeros((1, E), jnp.float32)
    for c in range(S // CH):
        sl = slice(c * CH, (c + 1) * CH)
        o0 = oh0[sl]
        o1 = oh1[sl]
        poff = jnp.dot(lstrict, o0 + o1,
                       preferred_element_type=jnp.float32) + base + offs
        pos0 = jnp.sum(o0 * poff, axis=1, keepdims=True)
        pos1 = jnp.sum(o1 * poff, axis=1, keepdims=True)
        p0_ref[sl, :] = jnp.broadcast_to(pos0, (CH, 8)).astype(jnp.int32)
        p1_ref[sl, :] = jnp.broadcast_to(pos1, (CH, 8)).astype(jnp.int32)
        base = base + jnp.sum(o0 + o1, axis=0, keepdims=True)

    starts = (lax.broadcasted_iota(jnp.int32, (128, E), 0) * TR).astype(
        jnp.float32)
    cmp = (jnp.broadcast_to(ends, (128, E)) <= starts).astype(jnp.int32)
    et = jnp.minimum(jnp.sum(cmp, axis=1, keepdims=True), laste)
    et_ref[...] = jnp.broadcast_to(et, (128, 128))
    nt_ref[...] = jnp.full((8, 128), (total / TR).astype(jnp.int32), jnp.int32)


def _gmm_kernel(et_ref, nt_ref, xs_ref, w1_ref, w2_ref, ys_ref):
    i = pl.program_id(0)

    @pl.when(i < nt_ref[0])
    def _():
        h = jnp.dot(xs_ref[...], w1_ref[0], preferred_element_type=jnp.float32)
        h = h * jax.nn.sigmoid(h)
        ys_ref[...] = jnp.dot(h, w2_ref[0], preferred_element_type=jnp.float32)


def _make_sc_kernels(S, D):
    TPW = S // NW
    HW = TPW // 2
    QW = TPW // 4
    mesh = plsc.VectorSubcoreMesh(core_axis_name="c", subcore_axis_name="s",
                                  num_cores=NC)

    @functools.partial(
        pl.kernel, mesh=mesh,
        out_type=jax.ShapeDtypeStruct((NP, D), jnp.float32),
        scratch_types=[
            pltpu.VMEM((HW,), jnp.int32),
            pltpu.VMEM((HW,), jnp.int32),
            pltpu.VMEM((HW,), jnp.int32),
            pltpu.VMEM((HW,), jnp.int32),
            pltpu.VMEM((HW, D), jnp.float32),
            pltpu.VMEM((HW, D), jnp.float32),
            pltpu.SemaphoreType.DMA,
            pltpu.SemaphoreType.DMA,
        ],
    )
    def dispatch(x_hbm, posr_hbm, xs_hbm, idx0a_v, idx1a_v, idx0b_v, idx1b_v,
                 rows_a, rows_b, sem_in, sem_out):
        wid = lax.axis_index("s") * NC + lax.axis_index("c")
        base = wid * TPW
        cpa = pltpu.async_copy(x_hbm.at[pl.ds(base, HW)], rows_a, sem_in)
        cpb = pltpu.async_copy(x_hbm.at[pl.ds(base + HW, HW)], rows_b, sem_in)
        pltpu.sync_copy(posr_hbm.at[wid, 0, pl.ds(0, HW)], idx0a_v)
        pltpu.sync_copy(posr_hbm.at[wid, 1, pl.ds(0, HW)], idx1a_v)
        pltpu.sync_copy(posr_hbm.at[wid, 0, pl.ds(HW, HW)], idx0b_v)
        pltpu.sync_copy(posr_hbm.at[wid, 1, pl.ds(HW, HW)], idx1b_v)
        cpa.wait()
        s0a = pltpu.async_copy(rows_a, xs_hbm.at[idx0a_v], sem_out)
        s1a = pltpu.async_copy(rows_a, xs_hbm.at[idx1a_v], sem_out)
        cpb.wait()
        s0b = pltpu.async_copy(rows_b, xs_hbm.at[idx0b_v], sem_out)
        s1b = pltpu.async_copy(rows_b, xs_hbm.at[idx1b_v], sem_out)
        s0a.wait()
        s1a.wait()
        s0b.wait()
        s1b.wait()

    @functools.partial(
        pl.kernel, mesh=mesh,
        out_type=jax.ShapeDtypeStruct((S, D), jnp.float32),
        scratch_types=[
            pltpu.VMEM((TPW,), jnp.int32),
            pltpu.VMEM((TPW,), jnp.int32),
            pltpu.VMEM((TPW, 2 * L), jnp.float32),
            pltpu.VMEM((QW, D), jnp.float32),
            pltpu.VMEM((QW, D), jnp.float32),
            pltpu.VMEM((QW, D), jnp.float32),
            pltpu.VMEM((QW, D), jnp.float32),
            pltpu.VMEM((QW, D), jnp.float32),
            pltpu.SemaphoreType.DMA,
        ],
    )
    def combine(ys_hbm, posr_hbm, wb_hbm, out_hbm,
                idx0_v, idx1_v, wb_v, yg0a, yg1a, yg0b, yg1b, out_v, sem):
        wid = lax.axis_index("s") * NC + lax.axis_index("c")
        base = wid * TPW
        pltpu.sync_copy(wb_hbm.at[pl.ds(base, TPW)], wb_v)
        pltpu.sync_copy(posr_hbm.at[wid, 0], idx0_v)
        pltpu.sync_copy(posr_hbm.at[wid, 1], idx1_v)
        bufs = [(yg0a, yg1a), (yg0b, yg1b)]

        def fire(sub):
            g0, g1 = bufs[sub % 2]
            s0 = sub * QW
            c0 = pltpu.async_copy(ys_hbm.at[idx0_v.at[pl.ds(s0, QW)]], g0, sem)
            c1 = pltpu.async_copy(ys_hbm.at[idx1_v.at[pl.ds(s0, QW)]], g1, sem)
            return c0, c1

        pend = fire(0)
        for sub in range(TPW // QW):
            s0 = sub * QW
            g0, g1 = bufs[sub % 2]
            pend[0].wait()
            pend[1].wait()
            if sub < TPW // QW - 1:
                pend = fire(sub + 1)

            def body(t, carry, s0=s0, g0=g0, g1=g1):
                w0 = wb_v[s0 + t, pl.ds(0, L)]
                w1 = wb_v[s0 + t, pl.ds(L, L)]
                for s2 in range(D // L):
                    slc = pl.ds(s2 * L, L)
                    out_v[t, slc] = w0 * g0[t, slc] + w1 * g1[t, slc]
                return carry

            lax.fori_loop(0, QW, body, 0)
            pltpu.sync_copy(out_v, out_hbm.at[pl.ds(base + s0, QW)])

    return dispatch, combine


def kernel(x, Wr, W1, W2):
    b, s, d = x.shape
    e, _, h = W1.shape
    xf = x.reshape(s, d)

    tpw = s // NW
    wb, p0o, p1o, eto, nto = pl.pallas_call(
        _router_meta_kernel,
        out_shape=[
            jax.ShapeDtypeStruct((s, 2 * L), jnp.float32),
            jax.ShapeDtypeStruct((s, 8), jnp.int32),
            jax.ShapeDtypeStruct((s, 8), jnp.int32),
            jax.ShapeDtypeStruct((128, 128), jnp.int32),
            jax.ShapeDtypeStruct((8, 128), jnp.int32),
        ],
    )(xf, Wr)

    posr = jnp.stack(
        [p0o[:, 0].reshape(NW, tpw), p1o[:, 0].reshape(NW, tpw)], axis=1)
    et = eto[:NT, 0]
    ntarr = nto[0, :1]

    dispatch, combine = _make_sc_kernels(s, d)
    xs = dispatch(xf, posr)

    grid_spec = pltpu.PrefetchScalarGridSpec(
        num_scalar_prefetch=2,
        grid=(NT,),
        in_specs=[
            pl.BlockSpec(
                (TR, d),
                lambda i, et_r, nt_r: (jnp.minimum(i, nt_r[0] - 1), 0)),
            pl.BlockSpec((1, d, h), lambda i, et_r, nt_r: (et_r[i], 0, 0)),
            pl.BlockSpec((1, h, d), lambda i, et_r, nt_r: (et_r[i], 0, 0)),
        ],
        out_specs=pl.BlockSpec(
            (TR, d), lambda i, et_r, nt_r: (jnp.minimum(i, nt_r[0] - 1), 0)),
    )
    ys = pl.pallas_call(
        _gmm_kernel,
        grid_spec=grid_spec,
        out_shape=jax.ShapeDtypeStruct((NP, d), jnp.float32),
    )(et, ntarr, xs, W1, W2)

    out = combine(ys, posr, wb)
    return out.reshape(b, s, d)
